# Initial kernel scaffold; baseline (speedup 1.0000x reference)
import functools

import jax
import jax.numpy as jnp
from jax import lax
from jax.experimental import pallas as pl
from jax.experimental.pallas import tpu as pltpu
from jax.experimental.pallas import tpu_sc as plsc

_NC = 2
_NS = 16
_NW = _NC * _NS
_LANES = 16


def _sc_gather(cat_flat, table_flat, num_fields):
    rows = cat_flat.shape[0]
    fv, d = table_flat.shape
    vocab = fv // num_fields
    per_w = rows // _NW
    nch = 8
    ch = per_w // nch
    vecs = per_w // _LANES

    mesh = plsc.VectorSubcoreMesh(core_axis_name="c", subcore_axis_name="s")

    @functools.partial(
        pl.kernel,
        mesh=mesh,
        out_type=jax.ShapeDtypeStruct((rows, d), jnp.float32),
        scratch_types=[
            pltpu.VMEM((per_w,), jnp.int32),
            pltpu.VMEM((2, ch, d), jnp.float32),
            pltpu.SemaphoreType.DMA,
            pltpu.SemaphoreType.DMA,
        ],
    )
    def k(cat_hbm, table_hbm, out_hbm, idx_v, rows_v, sem0, sem1):
        wid = lax.axis_index("s") * _NC + lax.axis_index("c")
        base = wid * per_w
        pltpu.sync_copy(cat_hbm.at[pl.ds(base, per_w)], idx_v)

        lane = lax.iota(jnp.int32, _LANES)

        def body(j, _):
            off = j * _LANES
            pos = off + lane
            fld = lax.rem(pos, num_fields)
            idx_v[pl.ds(off, _LANES)] = idx_v[pl.ds(off, _LANES)] + fld * vocab
            return 0

        lax.fori_loop(0, vecs, body, 0, unroll=4)

        sems = (sem0, sem1)
        prev = None
        for c in range(nch):
            buf = c % 2
            cp = pltpu.async_copy(
                table_hbm.at[idx_v.at[pl.ds(c * ch, ch)]], rows_v.at[buf],
                sems[buf]
            )
            if prev is not None:
                pcp, pc = prev
                pcp.wait()
                pltpu.sync_copy(
                    rows_v.at[pc % 2], out_hbm.at[pl.ds(base + pc * ch, ch)]
                )
            prev = (cp, c)
        pcp, pc = prev
        pcp.wait()
        pltpu.sync_copy(rows_v.at[pc % 2], out_hbm.at[pl.ds(base + pc * ch, ch)])

    return k(cat_flat, table_flat)


def _mlp_body(emb_ref, dense_ref, mean_ref, var_ref, w1e_ref, w1d_ref, b1_ref,
              woutt_ref, bout_ref, out_ref):
    normed = (dense_ref[...] - mean_ref[...]) * lax.rsqrt(var_ref[...] + 1e-6)
    h = jnp.dot(emb_ref[...], w1e_ref[...], preferred_element_type=jnp.float32)
    h = h + jnp.dot(normed, w1d_ref[...], preferred_element_type=jnp.float32)
    h = jnp.maximum(h + b1_ref[...], 0.0)
    o = jnp.sum(h * woutt_ref[...], axis=1, keepdims=True) + bout_ref[...]
    out_ref[...] = jax.nn.sigmoid(o)


def _tc_mlp(emb_flat, dense, mean, var, w1e, w1d, b1, woutt, bout):
    bsz, ed = emb_flat.shape
    nd = dense.shape[1]
    hid = w1e.shape[1]
    bm = 1024
    grid = (bsz // bm,)
    return pl.pallas_call(
        _mlp_body,
        grid=grid,
        in_specs=[
            pl.BlockSpec((bm, ed), lambda i: (i, 0)),
            pl.BlockSpec((bm, nd), lambda i: (i, 0)),
            pl.BlockSpec((1, nd), lambda i: (0, 0)),
            pl.BlockSpec((1, nd), lambda i: (0, 0)),
            pl.BlockSpec((ed, hid), lambda i: (0, 0)),
            pl.BlockSpec((nd, hid), lambda i: (0, 0)),
            pl.BlockSpec((1, hid), lambda i: (0, 0)),
            pl.BlockSpec((1, hid), lambda i: (0, 0)),
            pl.BlockSpec((1, 1), lambda i: (0, 0)),
        ],
        out_specs=pl.BlockSpec((bm, 1), lambda i: (i, 0)),
        out_shape=jax.ShapeDtypeStruct((bsz, 1), jnp.float32),
    )(emb_flat, dense, mean, var, w1e, w1d, b1, woutt, bout)


def kernel(cat_indices, dense_features, emb_tables, norm_mean, norm_var, W1,
           b1, W_out, b_out):
    b, f = cat_indices.shape
    _, v, d = emb_tables.shape
    cat_flat = cat_indices.reshape(b * f)
    table_flat = emb_tables.reshape(f * v, d)

    gathered = _sc_gather(cat_flat, table_flat, f)
    emb_flat = gathered.reshape(b, f * d)

    w1e = W1[: f * d]
    w1d = W1[f * d:]
    out = _tc_mlp(
        emb_flat,
        dense_features,
        norm_mean.reshape(1, -1),
        norm_var.reshape(1, -1),
        w1e,
        w1d,
        b1.reshape(1, -1),
        W_out.reshape(1, -1),
        b_out.reshape(1, 1),
    )
    return out


# speedup vs baseline: 1.3034x; 1.3034x over previous
import functools

import jax
import jax.numpy as jnp
from jax import lax
from jax.experimental import pallas as pl
from jax.experimental.pallas import tpu as pltpu
from jax.experimental.pallas import tpu_sc as plsc

_NC = 2
_NS = 16
_NW = _NC * _NS
_LANES = 16


def _sc_gather(cat_mat, table_flat, num_fields):
    rows = cat_mat.shape[0] * 128
    fv, d = table_flat.shape
    vocab = fv // num_fields
    per_w = rows // _NW
    nidx = per_w // 128
    gsz = 8
    ngrp = nidx // gsz
    grows = gsz * 128

    mesh = plsc.VectorSubcoreMesh(core_axis_name="c", subcore_axis_name="s")

    @functools.partial(
        pl.kernel,
        mesh=mesh,
        out_type=jax.ShapeDtypeStruct((rows, d), jnp.float32),
        compiler_params=pltpu.CompilerParams(use_tc_tiling_on_sc=False),
        scratch_types=[
            pltpu.VMEM((nidx, 128), jnp.int32),
            pltpu.VMEM((2, grows, d), jnp.float32),
            pltpu.SemaphoreType.DMA,
        ],
    )
    def k(cat_hbm, table_hbm, out_hbm, idx_v, rows_v, gsem):
        wid = lax.axis_index("s") * _NC + lax.axis_index("c")
        base = wid * per_w
        pltpu.sync_copy(cat_hbm.at[pl.ds(wid * nidx, nidx)], idx_v)

        lane = lax.iota(jnp.int32, _LANES)

        def body(t, _):
            j = lax.shift_right_logical(t, 3)
            kk = lax.bitwise_and(t, 7)
            pos = t * _LANES + lane
            fld = lax.rem(pos, num_fields)
            sl = (j, pl.ds(kk * _LANES, _LANES))
            idx_v[sl] = idx_v[sl] + fld * vocab
            return 0

        lax.fori_loop(0, per_w // _LANES, body, 0, unroll=4)

        def group(g, _):
            gbuf = lax.rem(g, 2)
            cps = []
            for bi in range(gsz):
                cps.append(pltpu.async_copy(
                    table_hbm.at[idx_v.at[g * gsz + bi]],
                    rows_v.at[gbuf, pl.ds(bi * 128, 128)],
                    gsem,
                ))
            @pl.when(g > 0)
            def _():
                pg = g - 1
                pltpu.sync_copy(
                    rows_v.at[lax.rem(pg, 2)],
                    out_hbm.at[pl.ds(base + pg * grows, grows)],
                )
            for cp in cps:
                cp.wait()
            return 0

        lax.fori_loop(0, ngrp, group, 0)
        pg = ngrp - 1
        pltpu.sync_copy(
            rows_v.at[pg % 2], out_hbm.at[pl.ds(base + pg * grows, grows)]
        )

    return k(cat_mat, table_flat)


def _mlp_body(emb_ref, dense_ref, mean_ref, var_ref, w1e_ref, w1d_ref, b1_ref,
              woutt_ref, bout_ref, out_ref):
    normed = (dense_ref[...] - mean_ref[...]) * lax.rsqrt(var_ref[...] + 1e-6)
    h = jnp.dot(emb_ref[...], w1e_ref[...], preferred_element_type=jnp.float32)
    h = h + jnp.dot(normed, w1d_ref[...], preferred_element_type=jnp.float32)
    h = jnp.maximum(h + b1_ref[...], 0.0)
    o = jnp.sum(h * woutt_ref[...], axis=1, keepdims=True) + bout_ref[...]
    out_ref[...] = jax.nn.sigmoid(o)


def _tc_mlp(emb_flat, dense, mean, var, w1e, w1d, b1, woutt, bout):
    bsz, ed = emb_flat.shape
    nd = dense.shape[1]
    hid = w1e.shape[1]
    bm = 1024
    grid = (bsz // bm,)
    return pl.pallas_call(
        _mlp_body,
        grid=grid,
        in_specs=[
            pl.BlockSpec((bm, ed), lambda i: (i, 0)),
            pl.BlockSpec((bm, nd), lambda i: (i, 0)),
            pl.BlockSpec((1, nd), lambda i: (0, 0)),
            pl.BlockSpec((1, nd), lambda i: (0, 0)),
            pl.BlockSpec((ed, hid), lambda i: (0, 0)),
            pl.BlockSpec((nd, hid), lambda i: (0, 0)),
            pl.BlockSpec((1, hid), lambda i: (0, 0)),
            pl.BlockSpec((1, hid), lambda i: (0, 0)),
            pl.BlockSpec((1, 1), lambda i: (0, 0)),
        ],
        out_specs=pl.BlockSpec((bm, 1), lambda i: (i, 0)),
        out_shape=jax.ShapeDtypeStruct((bsz, 1), jnp.float32),
    )(emb_flat, dense, mean, var, w1e, w1d, b1, woutt, bout)


def kernel(cat_indices, dense_features, emb_tables, norm_mean, norm_var, W1,
           b1, W_out, b_out):
    b, f = cat_indices.shape
    _, v, d = emb_tables.shape
    cat_mat = cat_indices.reshape(b * f // 128, 128)
    table_flat = emb_tables.reshape(f * v, d)

    gathered = _sc_gather(cat_mat, table_flat, f)
    emb_flat = gathered.reshape(b, f * d)

    w1e = W1[: f * d]
    w1d = W1[f * d:]
    out = _tc_mlp(
        emb_flat,
        dense_features,
        norm_mean.reshape(1, -1),
        norm_var.reshape(1, -1),
        w1e,
        w1d,
        b1.reshape(1, -1),
        W_out.reshape(1, -1),
        b_out.reshape(1, 1),
    )
    return out


# speedup vs baseline: 2.5766x; 1.9769x over previous
import functools

import jax
import jax.numpy as jnp
from jax import lax
from jax.experimental import pallas as pl
from jax.experimental.pallas import tpu as pltpu
from jax.experimental.pallas import tpu_sc as plsc

_NC = 2
_NS = 16
_NW = _NC * _NS
_LANES = 16


def _sc_gather2(cat_t, emb_tables):
    nj, nf, _ = cat_t.shape
    _, _, d = emb_tables.shape
    bsz = nj * 128
    per_w = bsz // _NW
    jpw = per_w // 128

    mesh = plsc.VectorSubcoreMesh(core_axis_name="c", subcore_axis_name="s")

    @functools.partial(
        pl.kernel,
        mesh=mesh,
        out_type=jax.ShapeDtypeStruct((bsz, nf * d), jnp.float32),
        compiler_params=pltpu.CompilerParams(use_tc_tiling_on_sc=False),
        scratch_types=[
            pltpu.VMEM((jpw, nf, 128), jnp.int32),
            pltpu.VMEM((2, per_w, d), jnp.float32),
            pltpu.SemaphoreType.DMA,
        ],
    )
    def k(cat_hbm, table_hbm, out_hbm, idx_v, fbuf, gsem):
        wid = lax.axis_index("s") * _NC + lax.axis_index("c")
        row0 = wid * per_w
        pltpu.sync_copy(cat_hbm.at[pl.ds(wid * jpw, jpw)], idx_v)

        def field(fi, _):
            buf = lax.rem(fi, 2)
            cps = []
            for j in range(jpw):
                cps.append(pltpu.async_copy(
                    table_hbm.at[fi].at[idx_v.at[j, fi]],
                    fbuf.at[buf, pl.ds(j * 128, 128)],
                    gsem,
                ))
            @pl.when(fi > 0)
            def _():
                pltpu.sync_copy(
                    fbuf.at[1 - buf],
                    out_hbm.at[pl.ds(row0, per_w), pl.ds((fi - 1) * d, d)],
                )
            for cp in cps:
                cp.wait()
            return 0

        lax.fori_loop(0, nf, field, 0)
        pltpu.sync_copy(
            fbuf.at[(nf - 1) % 2],
            out_hbm.at[pl.ds(row0, per_w), pl.ds((nf - 1) * d, d)],
        )

    return k(cat_t, emb_tables)


def _sc_gather(cat_mat, table_flat, num_fields):
    rows = cat_mat.shape[0] * 128
    fv, d = table_flat.shape
    vocab = fv // num_fields
    per_w = rows // _NW
    nidx = per_w // 128
    gsz = 8
    ngrp = nidx // gsz
    grows = gsz * 128

    mesh = plsc.VectorSubcoreMesh(core_axis_name="c", subcore_axis_name="s")

    @functools.partial(
        pl.kernel,
        mesh=mesh,
        out_type=jax.ShapeDtypeStruct((rows, d), jnp.float32),
        compiler_params=pltpu.CompilerParams(use_tc_tiling_on_sc=False),
        scratch_types=[
            pltpu.VMEM((nidx, 128), jnp.int32),
            pltpu.VMEM((2, grows, d), jnp.float32),
            pltpu.SemaphoreType.DMA,
        ],
    )
    def k(cat_hbm, table_hbm, out_hbm, idx_v, rows_v, gsem):
        wid = lax.axis_index("s") * _NC + lax.axis_index("c")
        base = wid * per_w
        pltpu.sync_copy(cat_hbm.at[pl.ds(wid * nidx, nidx)], idx_v)

        lane = lax.iota(jnp.int32, _LANES)

        def body(t, _):
            j = lax.shift_right_logical(t, 3)
            kk = lax.bitwise_and(t, 7)
            pos = t * _LANES + lane
            fld = lax.rem(pos, num_fields)
            sl = (j, pl.ds(kk * _LANES, _LANES))
            idx_v[sl] = idx_v[sl] + fld * vocab
            return 0

        lax.fori_loop(0, per_w // _LANES, body, 0, unroll=4)

        def group(g, _):
            gbuf = lax.rem(g, 2)
            cps = []
            for bi in range(gsz):
                cps.append(pltpu.async_copy(
                    table_hbm.at[idx_v.at[g * gsz + bi]],
                    rows_v.at[gbuf, pl.ds(bi * 128, 128)],
                    gsem,
                ))
            @pl.when(g > 0)
            def _():
                pg = g - 1
                pltpu.sync_copy(
                    rows_v.at[lax.rem(pg, 2)],
                    out_hbm.at[pl.ds(base + pg * grows, grows)],
                )
            for cp in cps:
                cp.wait()
            return 0

        lax.fori_loop(0, ngrp, group, 0)
        pg = ngrp - 1
        pltpu.sync_copy(
            rows_v.at[pg % 2], out_hbm.at[pl.ds(base + pg * grows, grows)]
        )

    return k(cat_mat, table_flat)


def _mlp_body(emb_ref, dense_ref, mean_ref, var_ref, w1e_ref, w1d_ref, b1_ref,
              woutt_ref, bout_ref, out_ref):
    normed = (dense_ref[...] - mean_ref[...]) * lax.rsqrt(var_ref[...] + 1e-6)
    h = jnp.dot(emb_ref[...], w1e_ref[...], preferred_element_type=jnp.float32)
    h = h + jnp.dot(normed, w1d_ref[...], preferred_element_type=jnp.float32)
    h = jnp.maximum(h + b1_ref[...], 0.0)
    o = jnp.sum(h * woutt_ref[...], axis=1, keepdims=True) + bout_ref[...]
    out_ref[...] = jax.nn.sigmoid(o)


def _tc_mlp(emb_flat, dense, mean, var, w1e, w1d, b1, woutt, bout):
    bsz, ed = emb_flat.shape
    nd = dense.shape[1]
    hid = w1e.shape[1]
    bm = 1024
    grid = (bsz // bm,)
    return pl.pallas_call(
        _mlp_body,
        grid=grid,
        in_specs=[
            pl.BlockSpec((bm, ed), lambda i: (i, 0)),
            pl.BlockSpec((bm, nd), lambda i: (i, 0)),
            pl.BlockSpec((1, nd), lambda i: (0, 0)),
            pl.BlockSpec((1, nd), lambda i: (0, 0)),
            pl.BlockSpec((ed, hid), lambda i: (0, 0)),
            pl.BlockSpec((nd, hid), lambda i: (0, 0)),
            pl.BlockSpec((1, hid), lambda i: (0, 0)),
            pl.BlockSpec((1, hid), lambda i: (0, 0)),
            pl.BlockSpec((1, 1), lambda i: (0, 0)),
        ],
        out_specs=pl.BlockSpec((bm, 1), lambda i: (i, 0)),
        out_shape=jax.ShapeDtypeStruct((bsz, 1), jnp.float32),
    )(emb_flat, dense, mean, var, w1e, w1d, b1, woutt, bout)


def kernel(cat_indices, dense_features, emb_tables, norm_mean, norm_var, W1,
           b1, W_out, b_out):
    b, f = cat_indices.shape
    _, v, d = emb_tables.shape
    cat_t = cat_indices.reshape(b // 128, 128, f).swapaxes(1, 2)
    emb_flat = _sc_gather2(cat_t, emb_tables)

    w1e = W1[: f * d]
    w1d = W1[f * d:]
    out = _tc_mlp(
        emb_flat,
        dense_features,
        norm_mean.reshape(1, -1),
        norm_var.reshape(1, -1),
        w1e,
        w1d,
        b1.reshape(1, -1),
        W_out.reshape(1, -1),
        b_out.reshape(1, 1),
    )
    return out


# speedup vs baseline: 18.0448x; 7.0032x over previous
import functools

import jax
import jax.numpy as jnp
from jax import lax
from jax.experimental import pallas as pl
from jax.experimental.pallas import tpu as pltpu
from jax.experimental.pallas import tpu_sc as plsc

_NC = 2
_NS = 16
_NW = _NC * _NS
_LANES = 16


def _sc_gather3(cat3, table4, dim):
    nplanes, vb, _ = table4.shape
    nf, ng, _ = cat3.shape
    per_t = nplanes // _NW
    qg = ng // 4

    mesh = plsc.VectorSubcoreMesh(core_axis_name="c", subcore_axis_name="s")

    @functools.partial(
        pl.kernel,
        mesh=mesh,
        out_type=jax.ShapeDtypeStruct((nplanes, ng, 128), jnp.float32),
        compiler_params=pltpu.CompilerParams(
            use_tc_tiling_on_sc=False, needs_layout_passes=False
        ),
        scratch_types=[
            pltpu.VMEM((vb, 128), jnp.float32),
            pltpu.VMEM((qg, 128), jnp.int32),
            pltpu.VMEM((qg, 128), jnp.float32),
        ],
    )
    def k(cat_hbm, table_hbm, out_hbm, plane_v, catv, outv):
        wid = lax.axis_index("s") * _NC + lax.axis_index("c")

        def plane(pi, _):
            p = wid * per_t + pi
            fi = lax.div(p, dim)
            pltpu.sync_copy(table_hbm.at[p], plane_v)
            for q in range(4):
                pltpu.sync_copy(cat_hbm.at[fi, pl.ds(q * qg, qg)], catv)

                def gat(i, _):
                    r = lax.shift_right_logical(i, 3)
                    cc = lax.bitwise_and(i, 7)
                    idx = catv[r, pl.ds(cc * _LANES, _LANES)]
                    hi = lax.shift_right_logical(idx, 7)
                    lo = lax.bitwise_and(idx, 127)
                    outv[r, pl.ds(cc * _LANES, _LANES)] = plsc.load_gather(
                        plane_v, [hi, lo]
                    )
                    return 0

                lax.fori_loop(0, qg * 8, gat, 0, unroll=8)
                pltpu.sync_copy(outv, out_hbm.at[p, pl.ds(q * qg, qg)])
            return 0

        lax.fori_loop(0, per_t, plane, 0)

    return k(cat3, table4)


def _mlp2_body(emb_ref, dense_ref, mean_ref, var_ref, w1e_ref, w1d_ref,
               b1_ref, woutt_ref, bout_ref, out_ref):
    normed = (dense_ref[...] - mean_ref[...]) * lax.rsqrt(var_ref[...] + 1e-6)
    hd = jnp.dot(normed, w1d_ref[...], preferred_element_type=jnp.float32)
    for rb in range(8):
        x = emb_ref[:, rb, :]
        h = lax.dot_general(x, w1e_ref[...], (((0,), (0,)), ((), ())),
                            preferred_element_type=jnp.float32)
        h = jnp.maximum(h + hd[rb * 128:(rb + 1) * 128, :] + b1_ref[...], 0.0)
        o = jnp.sum(h * woutt_ref[...], axis=1, keepdims=True) + bout_ref[...]
        out_ref[pl.ds(rb * 128, 128), :] = jax.nn.sigmoid(o)


def _tc_mlp2(emb3, dense, mean, var, w1e, w1d, b1, woutt, bout):
    npl, ng, _ = emb3.shape
    bsz, nd = dense.shape
    hid = w1e.shape[1]
    bm = 1024
    gb = bm // 128
    grid = (bsz // bm,)
    return pl.pallas_call(
        _mlp2_body,
        grid=grid,
        in_specs=[
            pl.BlockSpec((npl, gb, 128), lambda i: (0, i, 0)),
            pl.BlockSpec((bm, nd), lambda i: (i, 0)),
            pl.BlockSpec((1, nd), lambda i: (0, 0)),
            pl.BlockSpec((1, nd), lambda i: (0, 0)),
            pl.BlockSpec((npl, hid), lambda i: (0, 0)),
            pl.BlockSpec((nd, hid), lambda i: (0, 0)),
            pl.BlockSpec((1, hid), lambda i: (0, 0)),
            pl.BlockSpec((1, hid), lambda i: (0, 0)),
            pl.BlockSpec((1, 1), lambda i: (0, 0)),
        ],
        out_specs=pl.BlockSpec((bm, 1), lambda i: (i, 0)),
        out_shape=jax.ShapeDtypeStruct((bsz, 1), jnp.float32),
    )(emb3, dense, mean, var, w1e, w1d, b1, woutt, bout)


def kernel(cat_indices, dense_features, emb_tables, norm_mean, norm_var, W1,
           b1, W_out, b_out):
    b, f = cat_indices.shape
    _, v, d = emb_tables.shape
    vb = (v + 127) // 128
    table4 = jnp.pad(
        emb_tables.transpose(0, 2, 1), ((0, 0), (0, 0), (0, vb * 128 - v))
    ).reshape(f * d, vb, 128)
    cat3 = cat_indices.T.reshape(f, b // 128, 128)

    emb3 = _sc_gather3(cat3, table4, d)

    out = _tc_mlp2(
        emb3,
        dense_features,
        norm_mean.reshape(1, -1),
        norm_var.reshape(1, -1),
        W1[: f * d],
        W1[f * d:],
        b1.reshape(1, -1),
        W_out.reshape(1, -1),
        b_out.reshape(1, 1),
    )
    return out


# speedup vs baseline: 19.5406x; 1.0829x over previous
import functools

import jax
import jax.numpy as jnp
from jax import lax
from jax.experimental import pallas as pl
from jax.experimental.pallas import tpu as pltpu
from jax.experimental.pallas import tpu_sc as plsc

_NC = 2
_NS = 16
_NW = _NC * _NS
_LANES = 16


def _sc_gather3(cat3, table4, dim):
    nplanes, vb, _ = table4.shape
    nf, ng, _ = cat3.shape
    per_t = nplanes // _NW
    qg = ng // 4

    mesh = plsc.VectorSubcoreMesh(core_axis_name="c", subcore_axis_name="s")

    @functools.partial(
        pl.kernel,
        mesh=mesh,
        out_type=jax.ShapeDtypeStruct((nplanes, ng, 128), jnp.float32),
        compiler_params=pltpu.CompilerParams(
            use_tc_tiling_on_sc=False, needs_layout_passes=False
        ),
        scratch_types=[
            pltpu.VMEM((vb, 128), jnp.float32),
            pltpu.VMEM((ng, 128), jnp.int32),
            pltpu.VMEM((2, qg, 128), jnp.float32),
            pltpu.SemaphoreType.DMA,
            pltpu.SemaphoreType.DMA,
            pltpu.SemaphoreType.DMA,
            pltpu.SemaphoreType.DMA,
        ],
    )
    def k(cat_hbm, table_hbm, out_hbm, plane_v, catv, outv, sem_p, sem_c,
          sem_o0, sem_o1):
        wid = lax.axis_index("s") * _NC + lax.axis_index("c")
        sem_o = (sem_o0, sem_o1)

        def plane(pi, _):
            p = wid * per_t + pi
            fi = lax.div(p, dim)
            cp_p = pltpu.async_copy(table_hbm.at[p], plane_v, sem_p)
            cp_c = pltpu.async_copy(cat_hbm.at[fi], catv, sem_c)
            cp_p.wait()
            cp_c.wait()
            for q in range(4):
                buf = q % 2

                def drain():
                    pltpu.make_async_copy(
                        outv.at[buf], out_hbm.at[p, pl.ds(q * qg, qg)],
                        sem_o[buf],
                    ).wait()

                if q >= 2:
                    drain()
                else:
                    pl.when(pi > 0)(drain)

                def gat(i, _):
                    r = q * qg + lax.shift_right_logical(i, 3)
                    cc = lax.bitwise_and(i, 7)
                    idx = catv[r, pl.ds(cc * _LANES, _LANES)]
                    hi = lax.shift_right_logical(idx, 7)
                    lo = lax.bitwise_and(idx, 127)
                    outv[buf, lax.shift_right_logical(i, 3),
                         pl.ds(cc * _LANES, _LANES)] = plsc.load_gather(
                        plane_v, [hi, lo]
                    )
                    return 0

                lax.fori_loop(0, qg * 8, gat, 0, unroll=8)
                pltpu.async_copy(
                    outv.at[buf], out_hbm.at[p, pl.ds(q * qg, qg)], sem_o[buf]
                )
            return 0

        lax.fori_loop(0, per_t, plane, 0)
        for buf in range(2):
            pltpu.make_async_copy(
                outv.at[buf], out_hbm.at[0, pl.ds(0, qg)], sem_o[buf]
            ).wait()

    return k(cat3, table4)


def _mlp2_body(emb_ref, dense_ref, mean_ref, var_ref, w1e_ref, w1d_ref,
               b1_ref, woutt_ref, bout_ref, out_ref):
    normed = (dense_ref[...] - mean_ref[...]) * lax.rsqrt(var_ref[...] + 1e-6)
    hd = jnp.dot(normed, w1d_ref[...], preferred_element_type=jnp.float32)
    for rb in range(8):
        x = emb_ref[:, rb, :]
        h = lax.dot_general(x, w1e_ref[...], (((0,), (0,)), ((), ())),
                            preferred_element_type=jnp.float32)
        h = jnp.maximum(h + hd[rb * 128:(rb + 1) * 128, :] + b1_ref[...], 0.0)
        o = jnp.sum(h * woutt_ref[...], axis=1, keepdims=True) + bout_ref[...]
        out_ref[pl.ds(rb * 128, 128), :] = jax.nn.sigmoid(o)


def _tc_mlp2(emb3, dense, mean, var, w1e, w1d, b1, woutt, bout):
    npl, ng, _ = emb3.shape
    bsz, nd = dense.shape
    hid = w1e.shape[1]
    bm = 1024
    gb = bm // 128
    grid = (bsz // bm,)
    return pl.pallas_call(
        _mlp2_body,
        grid=grid,
        in_specs=[
            pl.BlockSpec((npl, gb, 128), lambda i: (0, i, 0)),
            pl.BlockSpec((bm, nd), lambda i: (i, 0)),
            pl.BlockSpec((1, nd), lambda i: (0, 0)),
            pl.BlockSpec((1, nd), lambda i: (0, 0)),
            pl.BlockSpec((npl, hid), lambda i: (0, 0)),
            pl.BlockSpec((nd, hid), lambda i: (0, 0)),
            pl.BlockSpec((1, hid), lambda i: (0, 0)),
            pl.BlockSpec((1, hid), lambda i: (0, 0)),
            pl.BlockSpec((1, 1), lambda i: (0, 0)),
        ],
        out_specs=pl.BlockSpec((bm, 1), lambda i: (i, 0)),
        out_shape=jax.ShapeDtypeStruct((bsz, 1), jnp.float32),
    )(emb3, dense, mean, var, w1e, w1d, b1, woutt, bout)


def kernel(cat_indices, dense_features, emb_tables, norm_mean, norm_var, W1,
           b1, W_out, b_out):
    b, f = cat_indices.shape
    _, v, d = emb_tables.shape
    vb = (v + 127) // 128
    table4 = jnp.pad(
        emb_tables, ((0, 0), (0, vb * 128 - v), (0, 0))
    ).transpose(0, 2, 1).reshape(f * d, vb, 128)
    cat3 = cat_indices.T.reshape(f, b // 128, 128)

    emb3 = _sc_gather3(cat3, table4, d)

    out = _tc_mlp2(
        emb3,
        dense_features,
        norm_mean.reshape(1, -1),
        norm_var.reshape(1, -1),
        W1[: f * d],
        W1[f * d:],
        b1.reshape(1, -1),
        W_out.reshape(1, -1),
        b_out.reshape(1, 1),
    )
    return out


# speedup vs baseline: 19.5915x; 1.0026x over previous
import functools

import jax
import jax.numpy as jnp
from jax import lax
from jax.experimental import pallas as pl
from jax.experimental.pallas import tpu as pltpu
from jax.experimental.pallas import tpu_sc as plsc

_NC = 2
_NS = 16
_NW = _NC * _NS
_LANES = 16


def _sc_gather3(cat3, table4, dim):
    nplanes, vb, _ = table4.shape
    nf, ng, _ = cat3.shape
    per_t = nplanes // _NW
    qg = ng // 4

    mesh = plsc.VectorSubcoreMesh(core_axis_name="c", subcore_axis_name="s")

    @functools.partial(
        pl.kernel,
        mesh=mesh,
        out_type=jax.ShapeDtypeStruct((nplanes, ng, 128), jnp.float32),
        compiler_params=pltpu.CompilerParams(
            use_tc_tiling_on_sc=False, needs_layout_passes=False
        ),
        scratch_types=[
            pltpu.VMEM((vb, 128), jnp.float32),
            pltpu.VMEM((ng, 128), jnp.int32),
            pltpu.VMEM((2, qg, 128), jnp.float32),
            pltpu.SemaphoreType.DMA,
            pltpu.SemaphoreType.DMA,
            pltpu.SemaphoreType.DMA,
            pltpu.SemaphoreType.DMA,
        ],
    )
    def k(cat_hbm, table_hbm, out_hbm, plane_v, catv, outv, sem_p, sem_c,
          sem_o0, sem_o1):
        wid = lax.axis_index("s") * _NC + lax.axis_index("c")
        sem_o = (sem_o0, sem_o1)

        def plane(pi, _):
            p = wid * per_t + pi
            fi = lax.div(p, dim)
            cp_p = pltpu.async_copy(table_hbm.at[p], plane_v, sem_p)
            cp_c = pltpu.async_copy(cat_hbm.at[fi], catv, sem_c)
            cp_p.wait()
            cp_c.wait()
            for q in range(4):
                buf = q % 2

                def drain():
                    pltpu.make_async_copy(
                        outv.at[buf], out_hbm.at[p, pl.ds(q * qg, qg)],
                        sem_o[buf],
                    ).wait()

                if q >= 2:
                    drain()
                else:
                    pl.when(pi > 0)(drain)

                def gat(r, _):
                    for cc in range(8):
                        idx = catv[q * qg + r, pl.ds(cc * _LANES, _LANES)]
                        hi = lax.shift_right_logical(idx, 7)
                        lo = lax.bitwise_and(idx, 127)
                        outv[buf, r, pl.ds(cc * _LANES, _LANES)] = (
                            plsc.load_gather(plane_v, [hi, lo])
                        )
                    return 0

                lax.fori_loop(0, qg, gat, 0, unroll=4)
                pltpu.async_copy(
                    outv.at[buf], out_hbm.at[p, pl.ds(q * qg, qg)], sem_o[buf]
                )
            return 0

        lax.fori_loop(0, per_t, plane, 0)
        for buf in range(2):
            pltpu.make_async_copy(
                outv.at[buf], out_hbm.at[0, pl.ds(0, qg)], sem_o[buf]
            ).wait()

    return k(cat3, table4)


def _mlp2_body(emb_ref, dense_ref, mean_ref, var_ref, w1e_ref, w1d_ref,
               b1_ref, woutt_ref, bout_ref, out_ref):
    normed = (dense_ref[...] - mean_ref[...]) * lax.rsqrt(var_ref[...] + 1e-6)
    hd = jnp.dot(normed, w1d_ref[...], preferred_element_type=jnp.float32)
    for rb in range(8):
        x = emb_ref[:, rb, :]
        h = lax.dot_general(x, w1e_ref[...], (((0,), (0,)), ((), ())),
                            preferred_element_type=jnp.float32)
        h = jnp.maximum(h + hd[rb * 128:(rb + 1) * 128, :] + b1_ref[...], 0.0)
        o = jnp.sum(h * woutt_ref[...], axis=1, keepdims=True) + bout_ref[...]
        out_ref[pl.ds(rb * 128, 128), :] = jax.nn.sigmoid(o)


def _tc_mlp2(emb3, dense, mean, var, w1e, w1d, b1, woutt, bout):
    npl, ng, _ = emb3.shape
    bsz, nd = dense.shape
    hid = w1e.shape[1]
    bm = 1024
    gb = bm // 128
    grid = (bsz // bm,)
    return pl.pallas_call(
        _mlp2_body,
        grid=grid,
        in_specs=[
            pl.BlockSpec((npl, gb, 128), lambda i: (0, i, 0)),
            pl.BlockSpec((bm, nd), lambda i: (i, 0)),
            pl.BlockSpec((1, nd), lambda i: (0, 0)),
            pl.BlockSpec((1, nd), lambda i: (0, 0)),
            pl.BlockSpec((npl, hid), lambda i: (0, 0)),
            pl.BlockSpec((nd, hid), lambda i: (0, 0)),
            pl.BlockSpec((1, hid), lambda i: (0, 0)),
            pl.BlockSpec((1, hid), lambda i: (0, 0)),
            pl.BlockSpec((1, 1), lambda i: (0, 0)),
        ],
        out_specs=pl.BlockSpec((bm, 1), lambda i: (i, 0)),
        out_shape=jax.ShapeDtypeStruct((bsz, 1), jnp.float32),
    )(emb3, dense, mean, var, w1e, w1d, b1, woutt, bout)


def kernel(cat_indices, dense_features, emb_tables, norm_mean, norm_var, W1,
           b1, W_out, b_out):
    b, f = cat_indices.shape
    _, v, d = emb_tables.shape
    vb = (v + 127) // 128
    table4 = jnp.pad(
        emb_tables, ((0, 0), (0, vb * 128 - v), (0, 0))
    ).transpose(0, 2, 1).reshape(f * d, vb, 128)
    cat3 = cat_indices.T.reshape(f, b // 128, 128)

    emb3 = _sc_gather3(cat3, table4, d)

    out = _tc_mlp2(
        emb3,
        dense_features,
        norm_mean.reshape(1, -1),
        norm_var.reshape(1, -1),
        W1[: f * d],
        W1[f * d:],
        b1.reshape(1, -1),
        W_out.reshape(1, -1),
        b_out.reshape(1, 1),
    )
    return out


# speedup vs baseline: 24.6987x; 1.2607x over previous
import functools

import jax
import jax.numpy as jnp
from jax import lax
from jax.experimental import pallas as pl
from jax.experimental.pallas import tpu as pltpu
from jax.experimental.pallas import tpu_sc as plsc

_NC = 2
_NS = 16
_NW = _NC * _NS
_LANES = 16


def _sc_gather3(cat3, table4, dim):
    nplanes, vb, _ = table4.shape
    nf, ng, _ = cat3.shape
    per_t = nplanes // _NW
    qg = ng // 4

    mesh = plsc.VectorSubcoreMesh(core_axis_name="c", subcore_axis_name="s")

    @functools.partial(
        pl.kernel,
        mesh=mesh,
        out_type=jax.ShapeDtypeStruct((nplanes, ng, 128), jnp.float32),
        compiler_params=pltpu.CompilerParams(
            use_tc_tiling_on_sc=False, needs_layout_passes=False
        ),
        scratch_types=[
            pltpu.VMEM((vb, 128), jnp.float32),
            pltpu.VMEM((ng, 128), jnp.int32),
            pltpu.VMEM((2, qg, 128), jnp.float32),
            pltpu.SemaphoreType.DMA,
            pltpu.SemaphoreType.DMA,
            pltpu.SemaphoreType.DMA,
            pltpu.SemaphoreType.DMA,
        ],
    )
    def k(cat_hbm, table_hbm, out_hbm, plane_v, catv, outv, sem_p, sem_c,
          sem_o0, sem_o1):
        wid = lax.axis_index("s") * _NC + lax.axis_index("c")
        sem_o = (sem_o0, sem_o1)

        def plane(pi, _):
            p = wid * per_t + pi
            fi = lax.div(p, dim)
            cp_p = pltpu.async_copy(table_hbm.at[p], plane_v, sem_p)
            cp_c = pltpu.async_copy(cat_hbm.at[fi], catv, sem_c)
            cp_p.wait()
            cp_c.wait()
            for q in range(4):
                buf = q % 2

                def drain():
                    pltpu.make_async_copy(
                        outv.at[buf], out_hbm.at[p, pl.ds(q * qg, qg)],
                        sem_o[buf],
                    ).wait()

                if q >= 2:
                    drain()
                else:
                    pl.when(pi > 0)(drain)

                @plsc.parallel_loop(0, qg, unroll=4)
                def _(r):
                    for cc in range(8):
                        idx = catv[q * qg + r, pl.ds(cc * _LANES, _LANES)]
                        hi = lax.shift_right_logical(idx, 7)
                        lo = lax.bitwise_and(idx, 127)
                        outv[buf, r, pl.ds(cc * _LANES, _LANES)] = (
                            plsc.load_gather(plane_v, [hi, lo])
                        )
                pltpu.async_copy(
                    outv.at[buf], out_hbm.at[p, pl.ds(q * qg, qg)], sem_o[buf]
                )
            return 0

        lax.fori_loop(0, per_t, plane, 0)
        for buf in range(2):
            pltpu.make_async_copy(
                outv.at[buf], out_hbm.at[0, pl.ds(0, qg)], sem_o[buf]
            ).wait()

    return k(cat3, table4)


def _mlp2_body(emb_ref, dense_ref, mean_ref, var_ref, w1e_ref, w1d_ref,
               b1_ref, woutt_ref, bout_ref, out_ref):
    normed = (dense_ref[...] - mean_ref[...]) * lax.rsqrt(var_ref[...] + 1e-6)
    hd = jnp.dot(normed, w1d_ref[...], preferred_element_type=jnp.float32)
    for rb in range(8):
        x = emb_ref[:, rb, :]
        h = lax.dot_general(x, w1e_ref[...], (((0,), (0,)), ((), ())),
                            preferred_element_type=jnp.float32)
        h = jnp.maximum(h + hd[rb * 128:(rb + 1) * 128, :] + b1_ref[...], 0.0)
        o = jnp.sum(h * woutt_ref[...], axis=1, keepdims=True) + bout_ref[...]
        out_ref[pl.ds(rb * 128, 128), :] = jax.nn.sigmoid(o)


def _tc_mlp2(emb3, dense, mean, var, w1e, w1d, b1, woutt, bout):
    npl, ng, _ = emb3.shape
    bsz, nd = dense.shape
    hid = w1e.shape[1]
    bm = 1024
    gb = bm // 128
    grid = (bsz // bm,)
    return pl.pallas_call(
        _mlp2_body,
        grid=grid,
        in_specs=[
            pl.BlockSpec((npl, gb, 128), lambda i: (0, i, 0)),
            pl.BlockSpec((bm, nd), lambda i: (i, 0)),
            pl.BlockSpec((1, nd), lambda i: (0, 0)),
            pl.BlockSpec((1, nd), lambda i: (0, 0)),
            pl.BlockSpec((npl, hid), lambda i: (0, 0)),
            pl.BlockSpec((nd, hid), lambda i: (0, 0)),
            pl.BlockSpec((1, hid), lambda i: (0, 0)),
            pl.BlockSpec((1, hid), lambda i: (0, 0)),
            pl.BlockSpec((1, 1), lambda i: (0, 0)),
        ],
        out_specs=pl.BlockSpec((bm, 1), lambda i: (i, 0)),
        out_shape=jax.ShapeDtypeStruct((bsz, 1), jnp.float32),
    )(emb3, dense, mean, var, w1e, w1d, b1, woutt, bout)


def kernel(cat_indices, dense_features, emb_tables, norm_mean, norm_var, W1,
           b1, W_out, b_out):
    b, f = cat_indices.shape
    _, v, d = emb_tables.shape
    vb = (v + 127) // 128
    table4 = jnp.pad(
        emb_tables, ((0, 0), (0, vb * 128 - v), (0, 0))
    ).transpose(0, 2, 1).reshape(f * d, vb, 128)
    cat3 = cat_indices.T.reshape(f, b // 128, 128)

    emb3 = _sc_gather3(cat3, table4, d)

    out = _tc_mlp2(
        emb3,
        dense_features,
        norm_mean.reshape(1, -1),
        norm_var.reshape(1, -1),
        W1[: f * d],
        W1[f * d:],
        b1.reshape(1, -1),
        W_out.reshape(1, -1),
        b_out.reshape(1, 1),
    )
    return out
